# Initial kernel scaffold; baseline (speedup 1.0000x reference)
import jax
import jax.numpy as jnp
from jax.experimental import pallas as pl


def kernel(x, edge_index, edge_weight, edge_features, adj_e, T, W1, b1, W2, b2, We, Wv, Wc1, bc1, Wc2, bc2):
    raise NotImplementedError("write your pallas kernel here")


# speedup vs baseline: 5.2523x; 5.2523x over previous
import functools

import jax
import jax.numpy as jnp
from jax import lax
from jax.experimental import pallas as pl
from jax.experimental.pallas import tpu as pltpu
from jax.experimental.pallas import tpu_sc as plsc

N = 10000
E = 320000
M = 2000
NC = 2
NS = 16
NW = NC * NS
C = 128
EP = 79 * NW * C
EPT = EP // NW
NCHUNK = EPT // C
ROWS_PT = 632
NACC = NS * ROWS_PT


def _make_sc_spmm(D, weighted):
    mesh = plsc.VectorSubcoreMesh(core_axis_name="c", subcore_axis_name="s")

    @functools.partial(
        pl.kernel,
        out_type=jax.ShapeDtypeStruct((NC * NACC, D), jnp.float32),
        mesh=mesh,
        compiler_params=pltpu.CompilerParams(use_tc_tiling_on_sc=False),
        scratch_types=[
            pltpu.VMEM((C,), jnp.int32),
            pltpu.VMEM((C,), jnp.int32),
            pltpu.VMEM((C,), jnp.float32),
            pltpu.VMEM((C, D), jnp.float32),
            pltpu.VMEM_SHARED((NACC, D), jnp.float32),
            pltpu.SemaphoreType.DMA,
        ],
    )
    def k(h_hbm, src_hbm, dst_hbm, w_hbm, zero_hbm, out_hbm,
          src_v, dst_v, w_v, rows_v, acc_sh, sem):
        cid = lax.axis_index("c")
        sid = lax.axis_index("s")
        wid = sid * NC + cid

        rs = pl.ds(sid * ROWS_PT, ROWS_PT)
        pltpu.sync_copy(zero_hbm.at[rs], acc_sh.at[rs])
        plsc.subcore_barrier()

        def chunk_body(g, _):
            base = wid * EPT + g * C
            pltpu.sync_copy(src_hbm.at[pl.ds(base, C)], src_v)
            pltpu.sync_copy(dst_hbm.at[pl.ds(base, C)], dst_v)
            if weighted:
                pltpu.sync_copy(w_hbm.at[pl.ds(base, C)], w_v)
            pltpu.async_copy(h_hbm.at[src_v], rows_v, sem).wait()
            if weighted:
                def scale_body(i, _):
                    w16 = w_v[pl.ds(i * 16, 16)]
                    for l in range(16):
                        wi = w16[l]
                        r = i * 16 + l
                        for j in range(D // 16):
                            sl = pl.ds(j * 16, 16)
                            rows_v[r, sl] = rows_v[r, sl] * wi
                    return 0
                lax.fori_loop(0, C // 16, scale_body, 0)
            pltpu.sync_copy(rows_v, acc_sh.at[dst_v], add=True)
            return 0

        lax.fori_loop(0, NCHUNK, chunk_body, 0)
        plsc.subcore_barrier()

        pltpu.sync_copy(acc_sh.at[rs],
                        out_hbm.at[pl.ds(cid * NACC + sid * ROWS_PT, ROWS_PT)])

    return k


_sc_spmm_64w = _make_sc_spmm(64, True)
_sc_spmm_32w = _make_sc_spmm(32, True)
_sc_spmm_32u = _make_sc_spmm(32, False)


def _lin1_body(x_ref, w_ref, b_ref, o_ref):
    o_ref[...] = (jnp.dot(x_ref[...], w_ref[...],
                          preferred_element_type=jnp.float32)
                  + b_ref[...])


def _tc_lin1(x, W1, b1):
    return pl.pallas_call(
        _lin1_body,
        out_shape=jax.ShapeDtypeStruct((N, 64), jnp.float32),
    )(x, W1, b1[None, :])


def _comb1_body(pa_ref, pb_ref, w_ref, b_ref, o_ref):
    h = jnp.maximum(pa_ref[...] + pb_ref[...], 0.0)
    o_ref[...] = (jnp.dot(h, w_ref[...], preferred_element_type=jnp.float32)
                  + b_ref[...])


def _tc_comb1(p, W2, b2):
    return pl.pallas_call(
        _comb1_body,
        out_shape=jax.ShapeDtypeStruct((N, 32), jnp.float32),
    )(p[:N], p[NACC:NACC + N], W2, b2[None, :])


def _comb2_body(pa_ref, pb_ref, o_ref):
    o_ref[...] = jnp.maximum(pa_ref[...] + pb_ref[...], 0.0)


def _tc_comb2(p):
    return pl.pallas_call(
        _comb2_body,
        out_shape=jax.ShapeDtypeStruct((N, 32), jnp.float32),
    )(p[:N], p[NACC:NACC + N])


def _enew_body(adj_ref, ef_ref, we_ref, o_ref):
    t = jnp.dot(ef_ref[...], we_ref[...], preferred_element_type=jnp.float32)
    o_ref[...] = jnp.maximum(
        jnp.dot(adj_ref[...], t, preferred_element_type=jnp.float32), 0.0)


def _tc_enew(adj_e, edge_features, We):
    return pl.pallas_call(
        _enew_body,
        out_shape=jax.ShapeDtypeStruct((M, 32), jnp.float32),
    )(adj_e, edge_features, We)


_NBLK = 10
_BR = N // _NBLK


def _final_body(pa_ref, pb_ref, t_ref, en_ref, wv_ref, wc1_ref, bc1_ref,
                wc2_ref, bc2_ref, o_ref):
    nf = (pa_ref[...] + pb_ref[...]) * (1.0 / float(E))
    shared = jnp.maximum(
        jnp.dot(nf, wv_ref[...], preferred_element_type=jnp.float32)
        + jnp.dot(t_ref[...], en_ref[...], preferred_element_type=jnp.float32),
        0.0)
    l1 = jnp.maximum(
        jnp.dot(shared, wc1_ref[...], preferred_element_type=jnp.float32)
        + bc1_ref[...], 0.0)
    logits = (jnp.dot(l1, wc2_ref[...], preferred_element_type=jnp.float32)
              + bc2_ref[...])
    m = jnp.max(logits, axis=1, keepdims=True)
    ex = jnp.exp(logits - m)
    o_ref[...] = ex / jnp.sum(ex, axis=1, keepdims=True)


def _tc_final(p3, T, e_new, Wv, Wc1, bc1, Wc2, bc2):
    grid = (_NBLK,)
    return pl.pallas_call(
        _final_body,
        grid=grid,
        in_specs=[
            pl.BlockSpec((_BR, 32), lambda i: (i, 0)),
            pl.BlockSpec((_BR, 32), lambda i: (i, 0)),
            pl.BlockSpec((_BR, M), lambda i: (i, 0)),
            pl.BlockSpec((M, 32), lambda i: (0, 0)),
            pl.BlockSpec((32, 32), lambda i: (0, 0)),
            pl.BlockSpec((32, 32), lambda i: (0, 0)),
            pl.BlockSpec((1, 32), lambda i: (0, 0)),
            pl.BlockSpec((32, 2), lambda i: (0, 0)),
            pl.BlockSpec((1, 2), lambda i: (0, 0)),
        ],
        out_specs=pl.BlockSpec((_BR, 2), lambda i: (i, 0)),
        out_shape=jax.ShapeDtypeStruct((N, 2), jnp.float32),
    )(p3[:N], p3[NACC:NACC + N], T, e_new, Wv, Wc1, bc1[None, :],
      Wc2, bc2[None, :])


def kernel(x, edge_index, edge_weight, edge_features, adj_e, T,
           W1, b1, W2, b2, We, Wv, Wc1, bc1, Wc2, bc2):
    pad = EP - E
    src = jnp.concatenate([edge_index[0], jnp.zeros((pad,), jnp.int32)])
    dst = jnp.concatenate([edge_index[1], jnp.full((pad,), N, jnp.int32)])
    w = jnp.concatenate([edge_weight, jnp.zeros((pad,), jnp.float32)])
    zero64 = jnp.zeros((NACC, 64), jnp.float32)
    zero32 = jnp.zeros((NACC, 32), jnp.float32)

    def rpad(a):
        return jnp.concatenate(
            [a, jnp.zeros((NACC - N, a.shape[1]), jnp.float32)])

    g1 = _tc_lin1(x, W1, b1)
    p1 = _sc_spmm_64w(rpad(g1), src, dst, w, zero64)
    g2 = _tc_comb1(p1, W2, b2)
    p2 = _sc_spmm_32w(rpad(g2), src, dst, w, zero32)
    h2 = _tc_comb2(p2)
    p3 = _sc_spmm_32u(rpad(h2), src, dst, w, zero32)
    e_new = _tc_enew(adj_e, edge_features, We)
    return _tc_final(p3, T, e_new, Wv, Wc1, bc1, Wc2, bc2)


# speedup vs baseline: 7.2012x; 1.3710x over previous
import functools

import jax
import jax.numpy as jnp
from jax import lax
from jax.experimental import pallas as pl
from jax.experimental.pallas import tpu as pltpu
from jax.experimental.pallas import tpu_sc as plsc

N = 10000
E = 320000
M = 2000
NC = 2
NS = 16
NW = NC * NS
C = 128
NCHUNK = 80
NBUF = 4
EPT = NCHUNK * C
EP = NW * EPT
ROWS_PT = 632
NACC = NS * ROWS_PT


def _make_sc_spmm(D, weighted):
    mesh = plsc.VectorSubcoreMesh(core_axis_name="c", subcore_axis_name="s")

    @functools.partial(
        pl.kernel,
        out_type=jax.ShapeDtypeStruct((NC * NACC, D), jnp.float32),
        mesh=mesh,
        compiler_params=pltpu.CompilerParams(use_tc_tiling_on_sc=False),
        scratch_types=[
            pltpu.VMEM((NCHUNK, C), jnp.int32),
            pltpu.VMEM((NCHUNK, C), jnp.int32),
            pltpu.VMEM((NCHUNK, C), jnp.float32),
            pltpu.VMEM((NBUF, C, D), jnp.float32),
            pltpu.VMEM_SHARED((NACC, D), jnp.float32),
            [pltpu.SemaphoreType.DMA] * NBUF,
            [pltpu.SemaphoreType.DMA] * NBUF,
            pltpu.SemaphoreType.DMA,
        ],
    )
    def k(h_hbm, src_hbm, dst_hbm, w_hbm, zero_hbm, out_hbm,
          src_v, dst_v, w_v, rows_v, acc_sh, gsems, ssems, esem):
        cid = lax.axis_index("c")
        sid = lax.axis_index("s")
        wid = sid * NC + cid

        e1 = pltpu.async_copy(src_hbm.at[wid], src_v, esem)
        e2 = pltpu.async_copy(dst_hbm.at[wid], dst_v, esem)
        if weighted:
            e3 = pltpu.async_copy(w_hbm.at[wid], w_v, esem)
        rs = pl.ds(sid * ROWS_PT, ROWS_PT)
        pltpu.sync_copy(zero_hbm.at[rs], acc_sh.at[rs])
        e1.wait()
        e2.wait()
        if weighted:
            e3.wait()
        plsc.subcore_barrier()

        def fire_gather(g, b):
            pltpu.async_copy(h_hbm.at[src_v.at[g]], rows_v.at[b], gsems[b])

        def wait_gather(b):
            pltpu.make_async_copy(h_hbm.at[src_v.at[0]], rows_v.at[b],
                                  gsems[b]).wait()

        def fire_scatter(g, b):
            pltpu.async_copy(rows_v.at[b], acc_sh.at[dst_v.at[g]], ssems[b],
                             add=True)

        def wait_scatter(b):
            pltpu.make_async_copy(rows_v.at[b], acc_sh.at[dst_v.at[0]],
                                  ssems[b]).wait()

        fire_gather(0, 0)
        fire_gather(1, 1)

        def iter_body(i, _):
            for j in range(NBUF):
                g = i * NBUF + j
                wait_gather(j)
                if weighted:
                    def scale_body(q, _):
                        w16 = w_v[g, pl.ds(q * 16, 16)]
                        for l in range(16):
                            wi = w16[l]
                            r = q * 16 + l
                            for f in range(D // 16):
                                sl = pl.ds(f * 16, 16)
                                rows_v[j, r, sl] = rows_v[j, r, sl] * wi
                        return 0
                    lax.fori_loop(0, C // 16, scale_body, 0)
                fire_scatter(g, j)
                b2 = (j + 2) % NBUF
                g2 = g + 2

                @pl.when(g2 >= NBUF)
                def _():
                    wait_scatter(b2)

                @pl.when(g2 < NCHUNK)
                def _():
                    fire_gather(g2, b2)
            return 0

        lax.fori_loop(0, NCHUNK // NBUF, iter_body, 0)
        wait_scatter((NCHUNK - 2) % NBUF)
        wait_scatter((NCHUNK - 1) % NBUF)
        plsc.subcore_barrier()

        pltpu.sync_copy(acc_sh.at[rs],
                        out_hbm.at[pl.ds(cid * NACC + sid * ROWS_PT, ROWS_PT)])

    return k


_sc_spmm_64w = _make_sc_spmm(64, True)
_sc_spmm_32w = _make_sc_spmm(32, True)
_sc_spmm_32u = _make_sc_spmm(32, False)


def _lin1_body(x_ref, w_ref, b_ref, o_ref):
    o_ref[...] = (jnp.dot(x_ref[...], w_ref[...],
                          preferred_element_type=jnp.float32)
                  + b_ref[...])


def _tc_lin1(x, W1, b1):
    return pl.pallas_call(
        _lin1_body,
        out_shape=jax.ShapeDtypeStruct((N, 64), jnp.float32),
    )(x, W1, b1[None, :])


def _comb1_body(pa_ref, pb_ref, w_ref, b_ref, o_ref):
    h = jnp.maximum(pa_ref[...] + pb_ref[...], 0.0)
    o_ref[...] = (jnp.dot(h, w_ref[...], preferred_element_type=jnp.float32)
                  + b_ref[...])


def _tc_comb1(p, W2, b2):
    return pl.pallas_call(
        _comb1_body,
        out_shape=jax.ShapeDtypeStruct((N, 32), jnp.float32),
    )(p[:N], p[NACC:NACC + N], W2, b2[None, :])


def _comb2_body(pa_ref, pb_ref, o_ref):
    o_ref[...] = jnp.maximum(pa_ref[...] + pb_ref[...], 0.0)


def _tc_comb2(p):
    return pl.pallas_call(
        _comb2_body,
        out_shape=jax.ShapeDtypeStruct((N, 32), jnp.float32),
    )(p[:N], p[NACC:NACC + N])


def _enew_body(adj_ref, ef_ref, we_ref, o_ref):
    t = jnp.dot(ef_ref[...], we_ref[...], preferred_element_type=jnp.float32)
    o_ref[...] = jnp.maximum(
        jnp.dot(adj_ref[...], t, preferred_element_type=jnp.float32), 0.0)


def _tc_enew(adj_e, edge_features, We):
    return pl.pallas_call(
        _enew_body,
        out_shape=jax.ShapeDtypeStruct((M, 32), jnp.float32),
    )(adj_e, edge_features, We)


_NBLK = 10
_BR = N // _NBLK


def _final_body(pa_ref, pb_ref, t_ref, en_ref, wv_ref, wc1_ref, bc1_ref,
                wc2_ref, bc2_ref, o_ref):
    nf = (pa_ref[...] + pb_ref[...]) * (1.0 / float(E))
    shared = jnp.maximum(
        jnp.dot(nf, wv_ref[...], preferred_element_type=jnp.float32)
        + jnp.dot(t_ref[...], en_ref[...], preferred_element_type=jnp.float32),
        0.0)
    l1 = jnp.maximum(
        jnp.dot(shared, wc1_ref[...], preferred_element_type=jnp.float32)
        + bc1_ref[...], 0.0)
    logits = (jnp.dot(l1, wc2_ref[...], preferred_element_type=jnp.float32)
              + bc2_ref[...])
    m = jnp.max(logits, axis=1, keepdims=True)
    ex = jnp.exp(logits - m)
    o_ref[...] = ex / jnp.sum(ex, axis=1, keepdims=True)


def _tc_final(p3, T, e_new, Wv, Wc1, bc1, Wc2, bc2):
    grid = (_NBLK,)
    return pl.pallas_call(
        _final_body,
        grid=grid,
        in_specs=[
            pl.BlockSpec((_BR, 32), lambda i: (i, 0)),
            pl.BlockSpec((_BR, 32), lambda i: (i, 0)),
            pl.BlockSpec((_BR, M), lambda i: (i, 0)),
            pl.BlockSpec((M, 32), lambda i: (0, 0)),
            pl.BlockSpec((32, 32), lambda i: (0, 0)),
            pl.BlockSpec((32, 32), lambda i: (0, 0)),
            pl.BlockSpec((1, 32), lambda i: (0, 0)),
            pl.BlockSpec((32, 2), lambda i: (0, 0)),
            pl.BlockSpec((1, 2), lambda i: (0, 0)),
        ],
        out_specs=pl.BlockSpec((_BR, 2), lambda i: (i, 0)),
        out_shape=jax.ShapeDtypeStruct((N, 2), jnp.float32),
    )(p3[:N], p3[NACC:NACC + N], T, e_new, Wv, Wc1, bc1[None, :],
      Wc2, bc2[None, :])


def kernel(x, edge_index, edge_weight, edge_features, adj_e, T,
           W1, b1, W2, b2, We, Wv, Wc1, bc1, Wc2, bc2):
    pad = EP - E
    src = jnp.concatenate(
        [edge_index[0], jnp.zeros((pad,), jnp.int32)]).reshape(NW, NCHUNK, C)
    dst = jnp.concatenate(
        [edge_index[1], jnp.full((pad,), N, jnp.int32)]).reshape(NW, NCHUNK, C)
    w = jnp.concatenate(
        [edge_weight, jnp.zeros((pad,), jnp.float32)]).reshape(NW, NCHUNK, C)
    zero64 = jnp.zeros((NACC, 64), jnp.float32)
    zero32 = jnp.zeros((NACC, 32), jnp.float32)

    def rpad(a):
        return jnp.concatenate(
            [a, jnp.zeros((NACC - N, a.shape[1]), jnp.float32)])

    g1 = _tc_lin1(x, W1, b1)
    p1 = _sc_spmm_64w(rpad(g1), src, dst, w, zero64)
    g2 = _tc_comb1(p1, W2, b2)
    p2 = _sc_spmm_32w(rpad(g2), src, dst, w, zero32)
    h2 = _tc_comb2(p2)
    p3 = _sc_spmm_32u(rpad(h2), src, dst, w, zero32)
    e_new = _tc_enew(adj_e, edge_features, We)
    return _tc_final(p3, T, e_new, Wv, Wc1, bc1, Wc2, bc2)
